# baseline (speedup 1.0000x reference)
import functools

import jax
import jax.numpy as jnp
from jax import lax
from jax.experimental import pallas as pl
from jax.experimental.pallas import tpu as pltpu
from jax.experimental.pallas import tpu_sc as plsc

_NC, _NS = 2, 16
_LANES = 16
_TC_ROWS = 32768


def _argmax_body(x_ref, preds_ref):
    x = x_ref[...]
    R, C = x.shape
    m = jnp.max(x, axis=1, keepdims=True)
    eqb = (x == m).astype(jnp.bfloat16)
    wexp = (191 - lax.broadcasted_iota(jnp.int32, (C, 1), 0)) << 23
    w = lax.bitcast_convert_type(wexp, jnp.float32).astype(jnp.bfloat16)
    srow = lax.dot_general(w, eqb, (((0,), (1,)), ((), ())),
                           preferred_element_type=jnp.float32)
    e = (lax.bitcast_convert_type(srow, jnp.int32) >> 23) - 127
    preds_ref[0] = (64 - e).astype(jnp.int32)


def _argmax(x2, t):
    nblk = x2.shape[0] // _TC_ROWS
    c = x2.shape[1]
    return pl.pallas_call(
        _argmax_body,
        grid=(nblk,),
        in_specs=[pl.BlockSpec((_TC_ROWS, c), lambda i: (i, 0))],
        out_specs=[pl.BlockSpec((1, 1, _TC_ROWS), lambda i: (i, 0, 0))],
        out_shape=[jax.ShapeDtypeStruct((nblk, 1, _TC_ROWS), jnp.int32)],
    )(x2)[0]


def _make_sc_decode(b, t, blank):
    mesh = plsc.VectorSubcoreMesh(core_axis_name="c", subcore_axis_name="s",
                                  num_cores=1)
    nit = t // _LANES

    @functools.partial(
        pl.kernel, mesh=mesh,
        compiler_params=pltpu.CompilerParams(needs_layout_passes=False),
        out_type=jax.ShapeDtypeStruct((b, t), jnp.int32),
        scratch_types=[
            pltpu.VMEM((t + 2 * _LANES,), jnp.int32),
            pltpu.VMEM((t,), jnp.int32),
            pltpu.SemaphoreType.DMA,
        ],
    )
    def decode(preds_hbm, out_hbm, prow, orow, sem):
        cid = lax.axis_index("c")
        sid = lax.axis_index("s")
        wid = sid + cid

        @pl.when(wid < b)
        def _():
            neg = jnp.full((_LANES,), -1, jnp.int32)
            prow[pl.ds(0, _LANES)] = neg
            rpb = _TC_ROWS // t
            cp = pltpu.async_copy(
                preds_hbm.at[wid // rpb, 0, pl.ds((wid % rpb) * t, t)],
                prow.at[pl.ds(8, t)], sem)

            def fill(i, acc):
                orow[pl.ds(i * _LANES, _LANES)] = neg
                return acc

            lax.fori_loop(0, nit, fill, jnp.int32(0))
            cp.wait()

            def body(i, base):
                cur = prow[pl.ds(8 + i * _LANES, _LANES)]
                prev = prow[pl.ds(7 + i * _LANES, _LANES)]
                km = ((cur != prev) & (cur != blank)).astype(jnp.int32)
                csum = jnp.cumsum(km)
                pos = csum + (base - 1)
                plsc.store_scatter(orow, [pos], cur, mask=km == 1)
                return base + csum[_LANES - 1]

            lax.fori_loop(0, nit, body, jnp.int32(0))
            pltpu.sync_copy(orow, out_hbm.at[wid])

    return decode


def kernel(y_pred):
    b, t, c = y_pred.shape
    x2 = y_pred.reshape(b * t, c)
    preds3 = _argmax(x2, t)
    out = _make_sc_decode(b, t, c - 1)(preds3)
    return out.astype(jnp.int64)


# speedup vs baseline: 1.0251x; 1.0251x over previous
import functools

import jax
import jax.numpy as jnp
from jax import lax
from jax.experimental import pallas as pl
from jax.experimental.pallas import tpu as pltpu
from jax.experimental.pallas import tpu_sc as plsc

_NC, _NS = 2, 16
_LANES = 16
_TC_ROWS = 16384


def _argmax_body(x_ref, preds_ref):
    x = x_ref[...]
    R, C = x.shape
    m = jnp.max(x, axis=1, keepdims=True)
    eqb = (x == m).astype(jnp.bfloat16)
    wexp = (191 - lax.broadcasted_iota(jnp.int32, (C, 1), 0)) << 23
    w = lax.bitcast_convert_type(wexp, jnp.float32).astype(jnp.bfloat16)
    srow = lax.dot_general(w, eqb, (((0,), (1,)), ((), ())),
                           preferred_element_type=jnp.float32)
    e = (lax.bitcast_convert_type(srow, jnp.int32) >> 23) - 127
    preds_ref[0] = (64 - e).astype(jnp.int32)


def _argmax(x2, t):
    nblk = x2.shape[0] // _TC_ROWS
    c = x2.shape[1]
    return pl.pallas_call(
        _argmax_body,
        grid=(nblk,),
        in_specs=[pl.BlockSpec((_TC_ROWS, c), lambda i: (i, 0))],
        out_specs=[pl.BlockSpec((1, 1, _TC_ROWS), lambda i: (i, 0, 0))],
        out_shape=[jax.ShapeDtypeStruct((nblk, 1, _TC_ROWS), jnp.int32)],
    )(x2)[0]


def _make_sc_decode(b, t, blank):
    mesh = plsc.VectorSubcoreMesh(core_axis_name="c", subcore_axis_name="s",
                                  num_cores=1)
    nit = t // _LANES

    @functools.partial(
        pl.kernel, mesh=mesh,
        compiler_params=pltpu.CompilerParams(needs_layout_passes=False),
        out_type=jax.ShapeDtypeStruct((b, t), jnp.int32),
        scratch_types=[
            pltpu.VMEM((t + 2 * _LANES,), jnp.int32),
            pltpu.VMEM((t,), jnp.int32),
            pltpu.SemaphoreType.DMA,
        ],
    )
    def decode(preds_hbm, out_hbm, prow, orow, sem):
        cid = lax.axis_index("c")
        sid = lax.axis_index("s")
        wid = sid + cid

        @pl.when(wid < b)
        def _():
            neg = jnp.full((_LANES,), -1, jnp.int32)
            prow[pl.ds(0, _LANES)] = neg
            rpb = _TC_ROWS // t
            cp = pltpu.async_copy(
                preds_hbm.at[wid // rpb, 0, pl.ds((wid % rpb) * t, t)],
                prow.at[pl.ds(8, t)], sem)

            def fill(i, acc):
                orow[pl.ds(i * _LANES, _LANES)] = neg
                return acc

            lax.fori_loop(0, nit, fill, jnp.int32(0))
            cp.wait()

            def body(i, base):
                cur = prow[pl.ds(8 + i * _LANES, _LANES)]
                prev = prow[pl.ds(7 + i * _LANES, _LANES)]
                km = ((cur != prev) & (cur != blank)).astype(jnp.int32)
                csum = jnp.cumsum(km)
                pos = csum + (base - 1)
                plsc.store_scatter(orow, [pos], cur, mask=km == 1)
                return base + csum[_LANES - 1]

            lax.fori_loop(0, nit, body, jnp.int32(0))
            pltpu.sync_copy(orow, out_hbm.at[wid])

    return decode


def kernel(y_pred):
    b, t, c = y_pred.shape
    x2 = y_pred.reshape(b * t, c)
    preds3 = _argmax(x2, t)
    out = _make_sc_decode(b, t, c - 1)(preds3)
    return out.astype(jnp.int64)
